# baseline (speedup 1.0000x reference)
import functools

import jax
import jax.numpy as jnp
from jax import lax
from jax.experimental import pallas as pl
from jax.experimental.pallas import tpu as pltpu
from jax.experimental.pallas import tpu_sc as plsc

NC = 2
NS = 16
CH = 125


def _make_sc_scatter(n, d, e):
    nw = NC * NS
    epw = e // nw
    chunks = epw // CH
    zrows = 128
    npad = -(-n // (NS * zrows)) * (NS * zrows)
    rows_per_tile = npad // NS
    assert epw % CH == 0 and chunks % 8 == 0 and rows_per_tile % zrows == 0

    mesh = plsc.VectorSubcoreMesh(core_axis_name="c", subcore_axis_name="s")

    passes = 2
    cpp = chunks // passes
    assert chunks % passes == 0 and cpp % 8 == 0

    @functools.partial(
        pl.kernel, mesh=mesh,
        out_type=[jax.ShapeDtypeStruct((npad, d), jnp.float32),
                  jax.ShapeDtypeStruct((npad, d), jnp.float32)],
        scratch_types=[
            pltpu.VMEM((cpp, CH), jnp.int32),
            pltpu.VMEM((cpp, CH), jnp.int32),
            pltpu.VMEM((CH, d), jnp.float32),
            pltpu.VMEM((CH, d), jnp.float32),
            pltpu.VMEM_SHARED((npad, d), jnp.float32),
            pltpu.SemaphoreType.DMA,
            pltpu.SemaphoreType.DMA,
            pltpu.SemaphoreType.DMA,
            pltpu.SemaphoreType.DMA,
        ])
    def sc_kernel(x_hbm, src_hbm, dst_hbm, z_hbm, out0_hbm, out1_hbm,
                  src_v, dst_v, g0, g1, acc, sem0, sem1, ssem0, ssem1):
        c = lax.axis_index("c")
        s = lax.axis_index("s")
        w = c * NS + s

        row0 = s * rows_per_tile
        for i in range(rows_per_tile // zrows):
            pltpu.sync_copy(z_hbm, acc.at[pl.ds(row0 + i * zrows, zrows)])
        plsc.subcore_barrier()

        for p in range(passes):
            base = w * chunks + p * cpp
            pltpu.sync_copy(src_hbm.at[pl.ds(base, cpp)], src_v)
            pltpu.sync_copy(dst_hbm.at[pl.ds(base, cpp)], dst_v)
            pltpu.async_copy(x_hbm.at[src_v.at[0]], g0, sem0)

            def body(j, carry):
                @pl.when(j % 2 == 0)
                def _():
                    pltpu.make_async_copy(x_hbm.at[src_v.at[j]], g0,
                                          sem0).wait()

                    @pl.when(j + 1 < cpp)
                    def _():
                        @pl.when(j > 0)
                        def _():
                            pltpu.make_async_copy(
                                g1, acc.at[dst_v.at[j]], ssem1).wait()
                        pltpu.async_copy(x_hbm.at[src_v.at[j + 1]], g1, sem1)
                    pltpu.async_copy(g0, acc.at[dst_v.at[j]], ssem0, add=True)

                @pl.when(j % 2 == 1)
                def _():
                    pltpu.make_async_copy(x_hbm.at[src_v.at[j]], g1,
                                          sem1).wait()

                    @pl.when(j + 1 < cpp)
                    def _():
                        pltpu.make_async_copy(
                            g0, acc.at[dst_v.at[j]], ssem0).wait()
                        pltpu.async_copy(x_hbm.at[src_v.at[j + 1]], g0, sem0)
                    pltpu.async_copy(g1, acc.at[dst_v.at[j]], ssem1, add=True)
                return carry

            lax.fori_loop(0, cpp, body, 0)
            pltpu.make_async_copy(g0, acc.at[dst_v.at[0]], ssem0).wait()
            pltpu.make_async_copy(g1, acc.at[dst_v.at[0]], ssem1).wait()
        plsc.subcore_barrier()

        @pl.when(c == 0)
        def _():
            pltpu.sync_copy(acc.at[pl.ds(row0, rows_per_tile)],
                            out0_hbm.at[pl.ds(row0, rows_per_tile)])

        @pl.when(c == 1)
        def _():
            pltpu.sync_copy(acc.at[pl.ds(row0, rows_per_tile)],
                            out1_hbm.at[pl.ds(row0, rows_per_tile)])

    return sc_kernel


def _make_sc_deg(n, e):
    nw = NC * NS
    epw = e // nw
    npad = -(-n // (NS * 128)) * (NS * 128)
    stripe = npad // NS
    assert epw % 16 == 0 and stripe % 16 == 0

    mesh = plsc.VectorSubcoreMesh(core_axis_name="c", subcore_axis_name="s")

    chunks = epw // CH
    zrows = 128
    rows_per_tile = npad // NS

    @functools.partial(
        pl.kernel, mesh=mesh,
        out_type=[jax.ShapeDtypeStruct((npad, 128), jnp.float32),
                  jax.ShapeDtypeStruct((npad, 128), jnp.float32)],
        scratch_types=[
            pltpu.VMEM((chunks, CH), jnp.int32),
            pltpu.VMEM((CH, 128), jnp.float32),
            pltpu.VMEM_SHARED((npad, 128), jnp.float32),
        ])
    def deg_kernel(dst_hbm, z_hbm, ones_hbm, out0_hbm, out1_hbm,
                   dst_v, ones_v, dacc):
        c = lax.axis_index("c")
        s = lax.axis_index("s")
        w = c * NS + s
        pltpu.sync_copy(ones_hbm, ones_v)
        row0 = s * rows_per_tile
        for i in range(rows_per_tile // zrows):
            pltpu.sync_copy(z_hbm, dacc.at[pl.ds(row0 + i * zrows, zrows)])
        plsc.subcore_barrier()
        pltpu.sync_copy(dst_hbm.at[pl.ds(w * chunks, chunks)], dst_v)

        def body(j, carry):
            pltpu.sync_copy(ones_v, dacc.at[dst_v.at[j]], add=True)
            return carry

        lax.fori_loop(0, chunks, body, 0)
        plsc.subcore_barrier()

        @pl.when(c == 0)
        def _():
            pltpu.sync_copy(dacc.at[pl.ds(row0, rows_per_tile)],
                            out0_hbm.at[pl.ds(row0, rows_per_tile)])

        @pl.when(c == 1)
        def _():
            pltpu.sync_copy(dacc.at[pl.ds(row0, rows_per_tile)],
                            out1_hbm.at[pl.ds(row0, rows_per_tile)])

    return deg_kernel


def _rgcn_dense_body(p0, p1, dg, x, W, R, b, o):
    deg = jnp.maximum(dg[...], 1.0)
    agg = (p0[...] + p1[...]) / deg
    h = (jnp.dot(agg, W[...], preferred_element_type=jnp.float32)
         + jnp.dot(x[...], R[...], preferred_element_type=jnp.float32)
         + b[...])
    o[...] = jnp.maximum(h, 0.0)


def _tc_layer1(p0, p1, dg, x, W, R, b, n, d, blk):
    grid = (n // blk,)
    row = lambda i: (i, 0)
    fixed = lambda i: (0, 0)
    return pl.pallas_call(
        _rgcn_dense_body,
        grid=grid,
        in_specs=[
            pl.BlockSpec((blk, d), row), pl.BlockSpec((blk, d), row),
            pl.BlockSpec((blk, 1), row),
            pl.BlockSpec((blk, d), row),
            pl.BlockSpec((d, d), fixed), pl.BlockSpec((d, d), fixed),
            pl.BlockSpec((1, d), fixed),
        ],
        out_specs=pl.BlockSpec((blk, d), row),
        out_shape=jax.ShapeDtypeStruct((n, d), jnp.float32),
    )(p0, p1, dg, x, W, R, b)


def _pool_body(p0, p1, dg, x, W, R, b, bat, psum, cnt, *, nb):
    i = pl.program_id(0)
    deg = jnp.maximum(dg[...], 1.0)
    agg = (p0[...] + p1[...]) / deg
    h = (jnp.dot(agg, W[...], preferred_element_type=jnp.float32)
         + jnp.dot(x[...], R[...], preferred_element_type=jnp.float32)
         + b[...])
    h2 = jnp.maximum(h, 0.0)
    bm = bat[...].reshape(1, -1)
    labels = lax.broadcasted_iota(jnp.int32, (nb, bm.shape[1]), 0)
    mask = (labels == bm).astype(jnp.float32)
    ps = jnp.dot(mask, h2, preferred_element_type=jnp.float32)
    ct = jnp.broadcast_to(jnp.sum(mask, axis=1, keepdims=True), psum.shape)

    @pl.when(i == 0)
    def _():
        psum[...] = ps
        cnt[...] = ct

    @pl.when(i != 0)
    def _():
        psum[...] += ps
        cnt[...] += ct


def _tc_layer2_pool(p0, p1, dg, h1, bat3, W, R, b, n, d, nb, blk):
    grid = (n // blk,)
    row = lambda i: (i, 0)
    fixed = lambda i: (0, 0)
    return pl.pallas_call(
        functools.partial(_pool_body, nb=nb),
        grid=grid,
        in_specs=[
            pl.BlockSpec((blk, d), row), pl.BlockSpec((blk, d), row),
            pl.BlockSpec((blk, 1), row),
            pl.BlockSpec((blk, d), row),
            pl.BlockSpec((d, d), fixed), pl.BlockSpec((d, d), fixed),
            pl.BlockSpec((1, d), fixed),
            pl.BlockSpec((1, 1, blk), lambda i: (i, 0, 0)),
        ],
        out_specs=[pl.BlockSpec((nb, d), fixed), pl.BlockSpec((nb, d), fixed)],
        out_shape=[jax.ShapeDtypeStruct((nb, d), jnp.float32),
                   jax.ShapeDtypeStruct((nb, d), jnp.float32)],
    )(p0, p1, dg, h1, W, R, b, bat3)


def _heads_body(psum, cnt, Wh, bh, Wc, bc, Wo, bo, logits, hidden, cell):
    pooled = psum[...] / jnp.maximum(cnt[...], 1.0)
    hid = jnp.dot(pooled, Wh[...], preferred_element_type=jnp.float32) + bh[...]
    cel = jnp.dot(pooled, Wc[...], preferred_element_type=jnp.float32) + bc[...]
    z = jnp.dot(hid, Wo[...], preferred_element_type=jnp.float32) + bo[...]
    m = jnp.max(z, axis=1, keepdims=True)
    lse = jnp.log(jnp.sum(jnp.exp(z - m), axis=1, keepdims=True)) + m
    logits[...] = z - lse
    hidden[...] = hid
    cell[...] = cel


def _tc_heads(psum, cnt, Wh, bh, Wc, bc, Wop, bop, nb, d, vp):
    return pl.pallas_call(
        _heads_body,
        out_shape=[jax.ShapeDtypeStruct((nb, vp), jnp.float32),
                   jax.ShapeDtypeStruct((nb, d), jnp.float32),
                   jax.ShapeDtypeStruct((nb, d), jnp.float32)],
    )(psum, cnt, Wh, bh, Wc, bc, Wop, bop)


def kernel(prev_symbol, x, edge_index, batch, W1, R1, b1, W2, R2, b2,
           Wh, bh, Wc, bc, Wo, bo):
    n, d = x.shape
    e = edge_index.shape[1]
    nb = prev_symbol.shape[0]
    v = bo.shape[0]
    vp = 128
    blk = 1000

    npad = -(-n // (NS * 128)) * (NS * 128)
    src2 = edge_index[0].reshape(e // CH, CH)
    dst2 = edge_index[1].reshape(e // CH, CH)
    zeros = jnp.zeros((128, d), jnp.float32)
    ones128 = jnp.ones((CH, 128), jnp.float32)
    bat3 = batch.reshape(n // blk, 1, blk)
    Wop = jnp.pad(Wo, ((0, 0), (0, vp - v)))
    bop = jnp.pad(bo, (0, vp - v), constant_values=-1e9).reshape(1, vp)

    sc_scatter = _make_sc_scatter(n, d, e)
    sc_deg = _make_sc_deg(n, e)

    dega, degb = sc_deg(dst2, zeros, ones128)
    dg = dega[:n, :1] + degb[:n, :1]
    a1, b1p = sc_scatter(x, src2, dst2, zeros)
    h1 = _tc_layer1(a1, b1p, dg, x,
                    W1, R1, b1.reshape(1, d), n, d, blk)
    a2, b2p = sc_scatter(h1, src2, dst2, zeros)
    psum, cnt = _tc_layer2_pool(a2, b2p, dg, h1,
                                bat3, W2, R2, b2.reshape(1, d), n, d, nb, blk)
    logits_p, hidden, cell = _tc_heads(psum, cnt, Wh, bh.reshape(1, d),
                                       Wc, bc.reshape(1, d), Wop, bop,
                                       nb, d, vp)
    return (logits_p[:, :v], hidden, cell)


# speedup vs baseline: 1.0294x; 1.0294x over previous
import functools

import jax
import jax.numpy as jnp
from jax import lax
from jax.experimental import pallas as pl
from jax.experimental.pallas import tpu as pltpu
from jax.experimental.pallas import tpu_sc as plsc

NC = 2
NS = 16
CH = 125


def _make_sc_scatter(n, d, e):
    nw = NC * NS
    epw = e // nw
    chunks = epw // CH
    zrows = 128
    npad = -(-n // (NS * zrows)) * (NS * zrows)
    rows_per_tile = npad // NS
    assert epw % CH == 0 and chunks % 8 == 0 and rows_per_tile % zrows == 0

    mesh = plsc.VectorSubcoreMesh(core_axis_name="c", subcore_axis_name="s")

    passes = 2
    cpp = chunks // passes
    assert chunks % passes == 0 and cpp % 8 == 0

    @functools.partial(
        pl.kernel, mesh=mesh,
        out_type=[jax.ShapeDtypeStruct((NC * npad, d), jnp.float32)],
        scratch_types=[
            pltpu.VMEM((cpp, CH), jnp.int32),
            pltpu.VMEM((cpp, CH), jnp.int32),
            pltpu.VMEM((CH, d), jnp.float32),
            pltpu.VMEM((CH, d), jnp.float32),
            pltpu.VMEM_SHARED((npad, d), jnp.float32),
            pltpu.SemaphoreType.DMA,
            pltpu.SemaphoreType.DMA,
        ])
    def sc_kernel(x_hbm, src_hbm, dst_hbm, z_hbm, out_hbm,
                  src_v, dst_v, g0, g1, acc, sem0, sem1):
        c = lax.axis_index("c")
        s = lax.axis_index("s")
        w = c * NS + s

        row0 = s * rows_per_tile
        for i in range(rows_per_tile // zrows):
            pltpu.sync_copy(z_hbm, acc.at[pl.ds(row0 + i * zrows, zrows)])
        plsc.subcore_barrier()

        for p in range(passes):
            base = w * chunks + p * cpp
            pltpu.sync_copy(src_hbm.at[pl.ds(base, cpp)], src_v)
            pltpu.sync_copy(dst_hbm.at[pl.ds(base, cpp)], dst_v)
            pltpu.async_copy(x_hbm.at[src_v.at[0]], g0, sem0)

            def body(j, carry):
                @pl.when(j % 2 == 0)
                def _():
                    pltpu.make_async_copy(x_hbm.at[src_v.at[j]], g0,
                                          sem0).wait()

                    @pl.when(j + 1 < cpp)
                    def _():
                        pltpu.async_copy(x_hbm.at[src_v.at[j + 1]], g1, sem1)
                    pltpu.sync_copy(g0, acc.at[dst_v.at[j]], add=True)

                @pl.when(j % 2 == 1)
                def _():
                    pltpu.make_async_copy(x_hbm.at[src_v.at[j]], g1,
                                          sem1).wait()

                    @pl.when(j + 1 < cpp)
                    def _():
                        pltpu.async_copy(x_hbm.at[src_v.at[j + 1]], g0, sem0)
                    pltpu.sync_copy(g1, acc.at[dst_v.at[j]], add=True)
                return carry

            lax.fori_loop(0, cpp, body, 0)
        plsc.subcore_barrier()

        pltpu.sync_copy(acc.at[pl.ds(row0, rows_per_tile)],
                        out_hbm.at[pl.ds(c * npad + row0, rows_per_tile)])

    return sc_kernel


def _make_sc_deg(n, e):
    nw = NC * NS
    epw = e // nw
    npad = -(-n // (NS * 128)) * (NS * 128)
    stripe = npad // NS
    assert epw % 16 == 0 and stripe % 16 == 0

    mesh = plsc.VectorSubcoreMesh(core_axis_name="c", subcore_axis_name="s")

    chunks = epw // CH
    zrows = 128
    rows_per_tile = npad // NS

    @functools.partial(
        pl.kernel, mesh=mesh,
        out_type=[jax.ShapeDtypeStruct((NC * npad, 128), jnp.float32)],
        scratch_types=[
            pltpu.VMEM((chunks, CH), jnp.int32),
            pltpu.VMEM((CH, 128), jnp.float32),
            pltpu.VMEM_SHARED((npad, 128), jnp.float32),
        ])
    def deg_kernel(dst_hbm, z_hbm, ones_hbm, out_hbm,
                   dst_v, ones_v, dacc):
        c = lax.axis_index("c")
        s = lax.axis_index("s")
        w = c * NS + s
        pltpu.sync_copy(ones_hbm, ones_v)
        row0 = s * rows_per_tile
        for i in range(rows_per_tile // zrows):
            pltpu.sync_copy(z_hbm, dacc.at[pl.ds(row0 + i * zrows, zrows)])
        plsc.subcore_barrier()
        pltpu.sync_copy(dst_hbm.at[pl.ds(w * chunks, chunks)], dst_v)

        def body(j, carry):
            pltpu.sync_copy(ones_v, dacc.at[dst_v.at[j]], add=True)
            return carry

        lax.fori_loop(0, chunks, body, 0)
        plsc.subcore_barrier()
        pltpu.sync_copy(dacc.at[pl.ds(row0, rows_per_tile)],
                        out_hbm.at[pl.ds(c * npad + row0, rows_per_tile)])

    return deg_kernel


def _rgcn_dense_body(p0, p1, dg, x, W, R, b, o):
    deg = jnp.maximum(dg[...], 1.0)
    agg = (p0[...] + p1[...]) / deg
    h = (jnp.dot(agg, W[...], preferred_element_type=jnp.float32)
         + jnp.dot(x[...], R[...], preferred_element_type=jnp.float32)
         + b[...])
    o[...] = jnp.maximum(h, 0.0)


def _tc_layer1(p0, p1, dg, x, W, R, b, n, d, blk):
    grid = (n // blk,)
    row = lambda i: (i, 0)
    fixed = lambda i: (0, 0)
    return pl.pallas_call(
        _rgcn_dense_body,
        grid=grid,
        in_specs=[
            pl.BlockSpec((blk, d), row), pl.BlockSpec((blk, d), row),
            pl.BlockSpec((blk, 1), row),
            pl.BlockSpec((blk, d), row),
            pl.BlockSpec((d, d), fixed), pl.BlockSpec((d, d), fixed),
            pl.BlockSpec((1, d), fixed),
        ],
        out_specs=pl.BlockSpec((blk, d), row),
        out_shape=jax.ShapeDtypeStruct((n, d), jnp.float32),
    )(p0, p1, dg, x, W, R, b)


def _pool_body(p0, p1, dg, x, W, R, b, bat, Wh, bh, Wc, bc, Wo, bo,
               logits, hidden, cell, psum, cnt, *, nb, ng):
    i = pl.program_id(0)
    deg = jnp.maximum(dg[...], 1.0)
    agg = (p0[...] + p1[...]) / deg
    h = (jnp.dot(agg, W[...], preferred_element_type=jnp.float32)
         + jnp.dot(x[...], R[...], preferred_element_type=jnp.float32)
         + b[...])
    h2 = jnp.maximum(h, 0.0)
    bm = bat[...].reshape(1, -1)
    labels = lax.broadcasted_iota(jnp.int32, (nb, bm.shape[1]), 0)
    mask = (labels == bm).astype(jnp.float32)
    ps = jnp.dot(mask, h2, preferred_element_type=jnp.float32)
    ct = jnp.broadcast_to(jnp.sum(mask, axis=1, keepdims=True), psum.shape)

    @pl.when(i == 0)
    def _():
        psum[...] = ps
        cnt[...] = ct

    @pl.when(i != 0)
    def _():
        psum[...] += ps
        cnt[...] += ct

    @pl.when(i == ng - 1)
    def _():
        pooled = psum[...] / jnp.maximum(cnt[...], 1.0)
        hid = (jnp.dot(pooled, Wh[...], preferred_element_type=jnp.float32)
               + bh[...])
        cel = (jnp.dot(pooled, Wc[...], preferred_element_type=jnp.float32)
               + bc[...])
        z = (jnp.dot(hid, Wo[...], preferred_element_type=jnp.float32)
             + bo[...])
        m = jnp.max(z, axis=1, keepdims=True)
        lse = jnp.log(jnp.sum(jnp.exp(z - m), axis=1, keepdims=True)) + m
        logits[...] = z - lse
        hidden[...] = hid
        cell[...] = cel


def _tc_layer2_pool(p0, p1, dg, h1, bat3, Wts, n, d, nb, vp, blk):
    W, R, b, Wh, bh, Wc, bc, Wop, bop = Wts
    ng = n // blk
    row = lambda i: (i, 0)
    fixed = lambda i: (0, 0)
    return pl.pallas_call(
        functools.partial(_pool_body, nb=nb, ng=ng),
        grid=(ng,),
        in_specs=[
            pl.BlockSpec((blk, d), row), pl.BlockSpec((blk, d), row),
            pl.BlockSpec((blk, 1), row),
            pl.BlockSpec((blk, d), row),
            pl.BlockSpec((d, d), fixed), pl.BlockSpec((d, d), fixed),
            pl.BlockSpec((1, d), fixed),
            pl.BlockSpec((1, 1, blk), lambda i: (i, 0, 0)),
            pl.BlockSpec((d, d), fixed), pl.BlockSpec((1, d), fixed),
            pl.BlockSpec((d, d), fixed), pl.BlockSpec((1, d), fixed),
            pl.BlockSpec((d, vp), fixed), pl.BlockSpec((1, vp), fixed),
        ],
        out_specs=[pl.BlockSpec((nb, vp), fixed),
                   pl.BlockSpec((nb, d), fixed),
                   pl.BlockSpec((nb, d), fixed)],
        out_shape=[jax.ShapeDtypeStruct((nb, vp), jnp.float32),
                   jax.ShapeDtypeStruct((nb, d), jnp.float32),
                   jax.ShapeDtypeStruct((nb, d), jnp.float32)],
        scratch_shapes=[pltpu.VMEM((nb, d), jnp.float32),
                        pltpu.VMEM((nb, d), jnp.float32)],
    )(p0, p1, dg, h1, W, R, b, bat3, Wh, bh, Wc, bc, Wop, bop)


def kernel(prev_symbol, x, edge_index, batch, W1, R1, b1, W2, R2, b2,
           Wh, bh, Wc, bc, Wo, bo):
    n, d = x.shape
    e = edge_index.shape[1]
    nb = prev_symbol.shape[0]
    v = bo.shape[0]
    vp = 128
    blk = 1000

    npad = -(-n // (NS * 128)) * (NS * 128)
    src2 = edge_index[0].reshape(e // CH, CH)
    dst2 = edge_index[1].reshape(e // CH, CH)
    zeros = jnp.zeros((128, d), jnp.float32)
    ones128 = jnp.ones((CH, 128), jnp.float32)
    bat3 = batch.reshape(n // blk, 1, blk)
    Wop = jnp.pad(Wo, ((0, 0), (0, vp - v)))
    bop = jnp.pad(bo, (0, vp - v), constant_values=-1e9).reshape(1, vp)

    sc_scatter = _make_sc_scatter(n, d, e)
    sc_deg = _make_sc_deg(n, e)

    (degp,) = sc_deg(dst2, zeros, ones128)
    dg = degp[:n, :1] + degp[npad:npad + n, :1]
    (agg1,) = sc_scatter(x, src2, dst2, zeros)
    h1 = _tc_layer1(agg1[:n], agg1[npad:npad + n], dg, x,
                    W1, R1, b1.reshape(1, d), n, d, blk)
    (agg2,) = sc_scatter(h1, src2, dst2, zeros)
    wts = (W2, R2, b2.reshape(1, d), Wh, bh.reshape(1, d),
           Wc, bc.reshape(1, d), Wop, bop)
    logits_p, hidden, cell = _tc_layer2_pool(
        agg2[:n], agg2[npad:npad + n], dg, h1, bat3, wts, n, d, nb, vp, blk)
    return (logits_p[:, :v], hidden, cell)


# speedup vs baseline: 1.0307x; 1.0013x over previous
import functools

import jax
import jax.numpy as jnp
from jax import lax
from jax.experimental import pallas as pl
from jax.experimental.pallas import tpu as pltpu
from jax.experimental.pallas import tpu_sc as plsc

NC = 2
NS = 16
CH = 125


def _make_sc_scatter(n, d, e):
    nw = NC * NS
    epw = e // nw
    chunks = epw // CH
    zrows = 128
    npad = -(-n // (NS * zrows)) * (NS * zrows)
    rows_per_tile = npad // NS
    assert epw % CH == 0 and chunks % 8 == 0 and rows_per_tile % zrows == 0

    mesh = plsc.VectorSubcoreMesh(core_axis_name="c", subcore_axis_name="s")

    passes = 2
    cpp = chunks // passes
    assert chunks % passes == 0 and cpp % 8 == 0

    @functools.partial(
        pl.kernel, mesh=mesh,
        out_type=[jax.ShapeDtypeStruct((NC * npad, d), jnp.float32)],
        scratch_types=[
            pltpu.VMEM((cpp, CH), jnp.int32),
            pltpu.VMEM((cpp, CH), jnp.int32),
            pltpu.VMEM((CH, d), jnp.float32),
            pltpu.VMEM((CH, d), jnp.float32),
            pltpu.VMEM_SHARED((npad, d), jnp.float32),
            pltpu.SemaphoreType.DMA,
            pltpu.SemaphoreType.DMA,
        ])
    def sc_kernel(x_hbm, src_hbm, dst_hbm, z_hbm, out_hbm,
                  src_v, dst_v, g0, g1, acc, sem0, sem1):
        c = lax.axis_index("c")
        s = lax.axis_index("s")
        w = c * NS + s

        row0 = s * rows_per_tile
        for i in range(rows_per_tile // zrows):
            pltpu.sync_copy(z_hbm, acc.at[pl.ds(row0 + i * zrows, zrows)])
        plsc.subcore_barrier()

        for p in range(passes):
            base = w * chunks + p * cpp
            pltpu.sync_copy(src_hbm.at[pl.ds(base, cpp)], src_v)
            pltpu.sync_copy(dst_hbm.at[pl.ds(base, cpp)], dst_v)
            pltpu.async_copy(x_hbm.at[src_v.at[0]], g0, sem0)

            def body(j, carry):
                @pl.when(j % 2 == 0)
                def _():
                    pltpu.make_async_copy(x_hbm.at[src_v.at[j]], g0,
                                          sem0).wait()

                    @pl.when(j + 1 < cpp)
                    def _():
                        pltpu.async_copy(x_hbm.at[src_v.at[j + 1]], g1, sem1)
                    pltpu.sync_copy(g0, acc.at[dst_v.at[j]], add=True)

                @pl.when(j % 2 == 1)
                def _():
                    pltpu.make_async_copy(x_hbm.at[src_v.at[j]], g1,
                                          sem1).wait()

                    @pl.when(j + 1 < cpp)
                    def _():
                        pltpu.async_copy(x_hbm.at[src_v.at[j + 1]], g0, sem0)
                    pltpu.sync_copy(g1, acc.at[dst_v.at[j]], add=True)
                return carry

            lax.fori_loop(0, cpp, body, 0)
        plsc.subcore_barrier()

        pltpu.sync_copy(acc.at[pl.ds(row0, rows_per_tile)],
                        out_hbm.at[pl.ds(c * npad + row0, rows_per_tile)])

    return sc_kernel


def _make_sc_deg(n, e):
    nw = NC * NS
    epw = e // nw
    npad = -(-n // (NS * 128)) * (NS * 128)
    stripe = npad // NS
    assert epw % 16 == 0 and stripe % 16 == 0

    mesh = plsc.VectorSubcoreMesh(core_axis_name="c", subcore_axis_name="s")

    chunks = epw // CH
    zrows = 128
    rows_per_tile = npad // NS

    @functools.partial(
        pl.kernel, mesh=mesh,
        out_type=[jax.ShapeDtypeStruct((NC * npad, 128), jnp.float32)],
        scratch_types=[
            pltpu.VMEM((chunks, CH), jnp.int32),
            pltpu.VMEM((CH, 128), jnp.float32),
            pltpu.VMEM_SHARED((npad, 128), jnp.float32),
            pltpu.SemaphoreType.DMA,
        ])
    def deg_kernel(dst_hbm, z_hbm, ones_hbm, out_hbm,
                   dst_v, ones_v, dacc, sem):
        c = lax.axis_index("c")
        s = lax.axis_index("s")
        w = c * NS + s
        pltpu.sync_copy(ones_hbm, ones_v)
        row0 = s * rows_per_tile
        for i in range(rows_per_tile // zrows):
            pltpu.sync_copy(z_hbm, dacc.at[pl.ds(row0 + i * zrows, zrows)])
        plsc.subcore_barrier()
        pltpu.sync_copy(dst_hbm.at[pl.ds(w * chunks, chunks)], dst_v)

        def body(j, carry):
            pltpu.async_copy(ones_v, dacc.at[dst_v.at[j]], sem, add=True)
            return carry

        lax.fori_loop(0, chunks, body, 0)

        def drain(j, carry):
            pltpu.make_async_copy(ones_v, dacc.at[dst_v.at[0]], sem).wait()
            return carry

        lax.fori_loop(0, chunks, drain, 0)
        plsc.subcore_barrier()
        pltpu.sync_copy(dacc.at[pl.ds(row0, rows_per_tile)],
                        out_hbm.at[pl.ds(c * npad + row0, rows_per_tile)])

    return deg_kernel


def _rgcn_dense_body(p0, p1, dg, x, W, R, b, o):
    deg = jnp.maximum(dg[...], 1.0)
    agg = (p0[...] + p1[...]) / deg
    h = (jnp.dot(agg, W[...], preferred_element_type=jnp.float32)
         + jnp.dot(x[...], R[...], preferred_element_type=jnp.float32)
         + b[...])
    o[...] = jnp.maximum(h, 0.0)


def _tc_layer1(p0, p1, dg, x, W, R, b, n, d, blk):
    grid = (n // blk,)
    row = lambda i: (i, 0)
    fixed = lambda i: (0, 0)
    return pl.pallas_call(
        _rgcn_dense_body,
        grid=grid,
        in_specs=[
            pl.BlockSpec((blk, d), row), pl.BlockSpec((blk, d), row),
            pl.BlockSpec((blk, 1), row),
            pl.BlockSpec((blk, d), row),
            pl.BlockSpec((d, d), fixed), pl.BlockSpec((d, d), fixed),
            pl.BlockSpec((1, d), fixed),
        ],
        out_specs=pl.BlockSpec((blk, d), row),
        out_shape=jax.ShapeDtypeStruct((n, d), jnp.float32),
    )(p0, p1, dg, x, W, R, b)


def _pool_body(p0, p1, dg, x, W, R, b, bat, Wh, bh, Wc, bc, Wo, bo,
               logits, hidden, cell, psum, cnt, *, nb, ng):
    i = pl.program_id(0)
    deg = jnp.maximum(dg[...], 1.0)
    agg = (p0[...] + p1[...]) / deg
    h = (jnp.dot(agg, W[...], preferred_element_type=jnp.float32)
         + jnp.dot(x[...], R[...], preferred_element_type=jnp.float32)
         + b[...])
    h2 = jnp.maximum(h, 0.0)
    bm = bat[...].reshape(1, -1)
    labels = lax.broadcasted_iota(jnp.int32, (nb, bm.shape[1]), 0)
    mask = (labels == bm).astype(jnp.float32)
    ps = jnp.dot(mask, h2, preferred_element_type=jnp.float32)
    ct = jnp.broadcast_to(jnp.sum(mask, axis=1, keepdims=True), psum.shape)

    @pl.when(i == 0)
    def _():
        psum[...] = ps
        cnt[...] = ct

    @pl.when(i != 0)
    def _():
        psum[...] += ps
        cnt[...] += ct

    @pl.when(i == ng - 1)
    def _():
        pooled = psum[...] / jnp.maximum(cnt[...], 1.0)
        hid = (jnp.dot(pooled, Wh[...], preferred_element_type=jnp.float32)
               + bh[...])
        cel = (jnp.dot(pooled, Wc[...], preferred_element_type=jnp.float32)
               + bc[...])
        z = (jnp.dot(hid, Wo[...], preferred_element_type=jnp.float32)
             + bo[...])
        m = jnp.max(z, axis=1, keepdims=True)
        lse = jnp.log(jnp.sum(jnp.exp(z - m), axis=1, keepdims=True)) + m
        logits[...] = z - lse
        hidden[...] = hid
        cell[...] = cel


def _tc_layer2_pool(p0, p1, dg, h1, bat3, Wts, n, d, nb, vp, blk):
    W, R, b, Wh, bh, Wc, bc, Wop, bop = Wts
    ng = n // blk
    row = lambda i: (i, 0)
    fixed = lambda i: (0, 0)
    return pl.pallas_call(
        functools.partial(_pool_body, nb=nb, ng=ng),
        grid=(ng,),
        in_specs=[
            pl.BlockSpec((blk, d), row), pl.BlockSpec((blk, d), row),
            pl.BlockSpec((blk, 1), row),
            pl.BlockSpec((blk, d), row),
            pl.BlockSpec((d, d), fixed), pl.BlockSpec((d, d), fixed),
            pl.BlockSpec((1, d), fixed),
            pl.BlockSpec((1, 1, blk), lambda i: (i, 0, 0)),
            pl.BlockSpec((d, d), fixed), pl.BlockSpec((1, d), fixed),
            pl.BlockSpec((d, d), fixed), pl.BlockSpec((1, d), fixed),
            pl.BlockSpec((d, vp), fixed), pl.BlockSpec((1, vp), fixed),
        ],
        out_specs=[pl.BlockSpec((nb, vp), fixed),
                   pl.BlockSpec((nb, d), fixed),
                   pl.BlockSpec((nb, d), fixed)],
        out_shape=[jax.ShapeDtypeStruct((nb, vp), jnp.float32),
                   jax.ShapeDtypeStruct((nb, d), jnp.float32),
                   jax.ShapeDtypeStruct((nb, d), jnp.float32)],
        scratch_shapes=[pltpu.VMEM((nb, d), jnp.float32),
                        pltpu.VMEM((nb, d), jnp.float32)],
    )(p0, p1, dg, h1, W, R, b, bat3, Wh, bh, Wc, bc, Wop, bop)


def kernel(prev_symbol, x, edge_index, batch, W1, R1, b1, W2, R2, b2,
           Wh, bh, Wc, bc, Wo, bo):
    n, d = x.shape
    e = edge_index.shape[1]
    nb = prev_symbol.shape[0]
    v = bo.shape[0]
    vp = 128
    blk = 1000

    npad = -(-n // (NS * 128)) * (NS * 128)
    src2 = edge_index[0].reshape(e // CH, CH)
    dst2 = edge_index[1].reshape(e // CH, CH)
    zeros = jnp.zeros((128, d), jnp.float32)
    ones128 = jnp.ones((CH, 128), jnp.float32)
    bat3 = batch.reshape(n // blk, 1, blk)
    Wop = jnp.pad(Wo, ((0, 0), (0, vp - v)))
    bop = jnp.pad(bo, (0, vp - v), constant_values=-1e9).reshape(1, vp)

    sc_scatter = _make_sc_scatter(n, d, e)
    sc_deg = _make_sc_deg(n, e)

    (degp,) = sc_deg(dst2, zeros, ones128)
    dg = degp[:n, :1] + degp[npad:npad + n, :1]
    (agg1,) = sc_scatter(x, src2, dst2, zeros)
    h1 = _tc_layer1(agg1[:n], agg1[npad:npad + n], dg, x,
                    W1, R1, b1.reshape(1, d), n, d, blk)
    (agg2,) = sc_scatter(h1, src2, dst2, zeros)
    wts = (W2, R2, b2.reshape(1, d), Wh, bh.reshape(1, d),
           Wc, bc.reshape(1, d), Wop, bop)
    logits_p, hidden, cell = _tc_layer2_pool(
        agg2[:n], agg2[npad:npad + n], dg, h1, bat3, wts, n, d, nb, vp, blk)
    return (logits_p[:, :v], hidden, cell)
